# baseline (speedup 1.0000x reference)
import functools

import jax
import jax.numpy as jnp
from jax import lax
from jax.experimental import pallas as pl
from jax.experimental.pallas import tpu as pltpu
from jax.experimental.pallas import tpu_sc as plsc

N_EMB = 100000
D = 128
N_TRIP = 100000

NC = 2
NS = 16
NW = NC * NS
BLK = 112
GRP = 4
NB_A = 36
NB_B = 20
NB_SUM = NB_A + NB_B
N_PAD = NS * NB_SUM * BLK
LANES = 16
CHUNKS = D // LANES

_mesh = plsc.VectorSubcoreMesh(core_axis_name="c", subcore_axis_name="s")


@functools.partial(
    pl.kernel,
    mesh=_mesh,
    out_type=jax.ShapeDtypeStruct((N_PAD * LANES,), jnp.float32),
    compiler_params=pltpu.CompilerParams(use_tc_tiling_on_sc=False),
    scratch_types=[
        pltpu.VMEM((2 * BLK,), jnp.int32),
        pltpu.VMEM((2 * BLK,), jnp.int32),
        pltpu.VMEM((2 * BLK,), jnp.int32),
        pltpu.VMEM((2 * BLK,), jnp.int32),
        pltpu.VMEM((2 * BLK,), jnp.int32),
        pltpu.VMEM((2 * BLK,), jnp.int32),
        pltpu.VMEM((BLK, D // 2), jnp.int32),
        pltpu.VMEM((BLK, D // 2), jnp.int32),
        pltpu.VMEM((BLK, D // 2), jnp.int32),
        pltpu.VMEM((BLK, D // 2), jnp.int32),
        pltpu.VMEM((BLK, D // 2), jnp.int32),
        pltpu.VMEM((BLK, D // 2), jnp.int32),
        pltpu.VMEM((GRP * BLK * LANES,), jnp.float32),
        pltpu.VMEM((GRP * BLK * LANES,), jnp.float32),
        pltpu.SemaphoreType.DMA,
        pltpu.SemaphoreType.DMA,
        pltpu.SemaphoreType.DMA,
        pltpu.SemaphoreType.DMA,
        pltpu.SemaphoreType.DMA,
        pltpu.SemaphoreType.DMA,
    ],
)
def _sc_partials(emb, t0, t1, t2, out,
                 ia0, it0, ii0, ia1, it1, ii1,
                 a0, tb0, ib0, a1, tb1, ib1,
                 xpa, xpb, semr0, semr1, si0, si1, semoa, semob):
    cbit = lax.axis_index("c")
    sid = lax.axis_index("s")
    base0 = (sid * NB_SUM + cbit * NB_A) * BLK

    idxs = ((ia0, it0, ii0, si0), (ia1, it1, ii1, si1))
    rows = ((a0, tb0, ib0, semr0), (a1, tb1, ib1, semr1))
    xps = ((xpa, semoa), (xpb, semob))

    def idx_fetch(pair, iset):
        ia, it, ii, si = idxs[iset]
        off = base0 + pair * (2 * BLK)
        pltpu.make_async_copy(t0.at[pl.ds(off, 2 * BLK)], ia, si).start()
        pltpu.make_async_copy(t1.at[pl.ds(off, 2 * BLK)], it, si).start()
        pltpu.make_async_copy(t2.at[pl.ds(off, 2 * BLK)], ii, si).start()

    def idx_wait(iset):
        ia, it, ii, si = idxs[iset]
        pltpu.make_async_copy(t0.at[pl.ds(base0, 2 * BLK)], ia, si).wait()
        pltpu.make_async_copy(t0.at[pl.ds(base0, 2 * BLK)], it, si).wait()
        pltpu.make_async_copy(t0.at[pl.ds(base0, 2 * BLK)], ii, si).wait()

    H = BLK // 2

    def fire(rset, iset, half):
        ia, it, ii, _ = idxs[iset]
        ab, tb, ib, semr = rows[rset]
        sl0 = pl.ds(half * BLK, H)
        sl1 = pl.ds(half * BLK + H, H)
        pltpu.make_async_copy(emb.at[ia.at[sl0]], ab.at[pl.ds(0, H), :], semr).start()
        pltpu.make_async_copy(emb.at[it.at[sl0]], tb.at[pl.ds(0, H), :], semr).start()
        pltpu.make_async_copy(emb.at[ii.at[sl0]], ib.at[pl.ds(0, H), :], semr).start()
        pltpu.make_async_copy(emb.at[ia.at[sl1]], ab.at[pl.ds(H, H), :], semr).start()
        pltpu.make_async_copy(emb.at[it.at[sl1]], tb.at[pl.ds(H, H), :], semr).start()
        pltpu.make_async_copy(emb.at[ii.at[sl1]], ib.at[pl.ds(H, H), :], semr).start()

    def drain(rset):
        ia, _, _, _ = idxs[0]
        ab, tb, ib, semr = rows[rset]
        sl = pl.ds(0, H)
        for dst in (ab, tb, ib):
            pltpu.make_async_copy(
                emb.at[ia.at[sl]], dst.at[pl.ds(0, H), :], semr).wait()
            pltpu.make_async_copy(
                emb.at[ia.at[sl]], dst.at[pl.ds(H, H), :], semr).wait()

    def out_start(q, xset):
        xp, semo = xps[xset]
        off = (base0 + q * (GRP * BLK)) * LANES
        pltpu.make_async_copy(xp, out.at[pl.ds(off, GRP * BLK * LANES)], semo).start()

    def out_wait(xset):
        xp, semo = xps[xset]
        pltpu.make_async_copy(
            xp, out.at[pl.ds(base0 * LANES, GRP * BLK * LANES)], semo).wait()

    def compute(blk, rset, xset, xrow):
        ab, tb, ib, _ = rows[rset]
        xp, _ = xps[xset]

        def triplet(j, carry):
            acc1 = jnp.zeros((LANES,), jnp.float32)
            acc12 = jnp.zeros((LANES,), jnp.float32)
            acc2 = jnp.zeros((LANES,), jnp.float32)
            shift16 = jnp.full((LANES,), 16, jnp.int32)

            def halves(w):
                lo = lax.bitcast_convert_type(
                    lax.shift_left(w, shift16), jnp.float32)
                hi = lax.bitcast_convert_type(
                    lax.shift_left(lax.shift_right_logical(w, shift16),
                                   shift16), jnp.float32)
                return lo, hi

            for c in range(D // 32):
                sl = pl.ds(c * LANES, LANES)
                al, ah = halves(ab[j, sl])
                tl, th = halves(tb[j, sl])
                il, ih = halves(ib[j, sl])
                d1l = tl - al
                d1h = th - ah
                d2l = il - al
                d2h = ih - ah
                acc1 = acc1 + d1l * d1l + d1h * d1h
                acc12 = acc12 + d1l * d2l + d1h * d2h
                acc2 = acc2 + d2l * d2l + d2h * d2h
            xp[pl.ds((xrow + j) * LANES, LANES)] = 0.5 * acc1 + acc12 - acc2
            return carry

        lax.fori_loop(0, BLK, triplet, 0)

    def schedule(nb):
        npair = nb // 2
        idx_fetch(0, 0)
        idx_fetch(1, 1)
        idx_wait(0)
        fire(0, 0, 0)
        fire(1, 0, 1)
        drain(0)
        compute(0, 0, 0, 0 * BLK)
        idx_wait(1)
        fire(0, 1, 0)
        drain(1)
        idx_fetch(2, 0)
        compute(1, 1, 0, 1 * BLK)
        fire(1, 1, 1)
        drain(0)
        compute(2, 0, 0, 2 * BLK)
        idx_wait(0)
        fire(0, 0, 0)
        drain(1)
        idx_fetch(3, 1)
        compute(3, 1, 0, 3 * BLK)
        out_start(0, 0)

        def outer(t, carry):
            q1 = 2 * t + 1
            b0 = q1 * GRP

            @pl.when(t > 0)
            def _():
                out_wait(1)

            fire(1, 0, 1)
            drain(0)
            compute(b0 + 0, 0, 1, 0 * BLK)
            idx_wait(1)
            fire(0, 1, 0)
            drain(1)
            idx_fetch(4 * t + 4, 0)
            compute(b0 + 1, 1, 1, 1 * BLK)
            fire(1, 1, 1)
            drain(0)
            compute(b0 + 2, 0, 1, 2 * BLK)
            idx_wait(0)
            fire(0, 0, 0)
            drain(1)
            idx_fetch(4 * t + 5, 1)
            compute(b0 + 3, 1, 1, 3 * BLK)
            out_start(q1, 1)

            q2 = q1 + 1
            c0 = q2 * GRP
            out_wait(0)
            fire(1, 0, 1)
            drain(0)
            compute(c0 + 0, 0, 0, 0 * BLK)
            idx_wait(1)
            fire(0, 1, 0)
            drain(1)

            @pl.when(4 * t + 6 < npair)
            def _():
                idx_fetch(4 * t + 6, 0)

            compute(c0 + 1, 1, 0, 1 * BLK)
            fire(1, 1, 1)
            drain(0)
            compute(c0 + 2, 0, 0, 2 * BLK)
            @pl.when(c0 + 4 < nb)
            def _():
                idx_wait(0)
                fire(0, 0, 0)

            drain(1)

            @pl.when(4 * t + 7 < npair)
            def _():
                idx_fetch(4 * t + 7, 1)

            compute(c0 + 3, 1, 0, 3 * BLK)
            out_start(q2, 0)
            return carry

        lax.fori_loop(0, (nb // GRP - 1) // 2, outer, 0)
        out_wait(1)
        out_wait(0)

    @pl.when(cbit == 0)
    def _():
        schedule(NB_A)

    @pl.when(cbit == 1)
    def _():
        schedule(NB_B)


def _tc_reduce_body(x_ref, out_ref):
    x = x_ref[...].reshape(N_PAD // 16, 256)
    g = lax.broadcasted_iota(jnp.int32, (256, 16), 0) // 16 == \
        lax.broadcasted_iota(jnp.int32, (256, 16), 1)
    s = jax.lax.dot_general(x, g.astype(jnp.float32),
                            (((1,), (0,)), ((), ())),
                            preferred_element_type=jnp.float32)
    sp = jnp.maximum(s, 0.0) + jnp.log1p(jnp.exp(-jnp.abs(s)))
    ids = lax.broadcasted_iota(jnp.int32, (N_PAD // 16, 16), 0) * 16 + \
        lax.broadcasted_iota(jnp.int32, (N_PAD // 16, 16), 1)
    total = jnp.sum(jnp.where(ids < N_TRIP, sp, 0.0), keepdims=True)
    out_ref[...] = total.reshape(1, 1) / N_TRIP


_tc_reduce = pl.pallas_call(
    _tc_reduce_body,
    out_shape=jax.ShapeDtypeStruct((1, 1), jnp.float32),
)


def kernel(inputs, targets, T):
    del targets
    bank = lax.bitcast_convert_type(
        inputs.astype(jnp.bfloat16).reshape(N_EMB, D // 2, 2), jnp.int32)
    t_pad = jnp.pad(T, ((0, 0), (0, N_PAD - T.shape[1])))
    xpart = _sc_partials(bank, t_pad[0], t_pad[1], t_pad[2])
    return _tc_reduce(xpart)[0, 0]


# speedup vs baseline: 4.1820x; 4.1820x over previous
import functools

import jax
import jax.numpy as jnp
from jax import lax
from jax.experimental import pallas as pl
from jax.experimental.pallas import tpu as pltpu
from jax.experimental.pallas import tpu_sc as plsc

N_EMB = 100000
D = 128
N_TRIP = 100000

NC = 2
NS = 16
NW = NC * NS
BLK = 112
GRP = 4
NB_A = 36
NB_B = 20
NB_SUM = NB_A + NB_B
N_PAD = NS * NB_SUM * BLK
LANES = 16
CHUNKS = D // LANES

_mesh = plsc.VectorSubcoreMesh(core_axis_name="c", subcore_axis_name="s")


@functools.partial(
    pl.kernel,
    mesh=_mesh,
    out_type=jax.ShapeDtypeStruct((N_PAD * LANES,), jnp.float32),
    scratch_types=[
        pltpu.VMEM((2 * BLK,), jnp.int32),
        pltpu.VMEM((2 * BLK,), jnp.int32),
        pltpu.VMEM((2 * BLK,), jnp.int32),
        pltpu.VMEM((2 * BLK,), jnp.int32),
        pltpu.VMEM((2 * BLK,), jnp.int32),
        pltpu.VMEM((2 * BLK,), jnp.int32),
        pltpu.VMEM((BLK, D), jnp.float32),
        pltpu.VMEM((BLK, D), jnp.float32),
        pltpu.VMEM((BLK, D), jnp.float32),
        pltpu.VMEM((BLK, D), jnp.float32),
        pltpu.VMEM((BLK, D), jnp.float32),
        pltpu.VMEM((BLK, D), jnp.float32),
        pltpu.VMEM((GRP * BLK * LANES,), jnp.float32),
        pltpu.VMEM((GRP * BLK * LANES,), jnp.float32),
        pltpu.SemaphoreType.DMA,
        pltpu.SemaphoreType.DMA,
        pltpu.SemaphoreType.DMA,
        pltpu.SemaphoreType.DMA,
        pltpu.SemaphoreType.DMA,
        pltpu.SemaphoreType.DMA,
    ],
)
def _sc_partials(emb, t0, t1, t2, out,
                 ia0, it0, ii0, ia1, it1, ii1,
                 a0, tb0, ib0, a1, tb1, ib1,
                 xpa, xpb, semr0, semr1, si0, si1, semoa, semob):
    cbit = lax.axis_index("c")
    sid = lax.axis_index("s")
    base0 = (sid * NB_SUM + cbit * NB_A) * BLK

    idxs = ((ia0, it0, ii0, si0), (ia1, it1, ii1, si1))
    rows = ((a0, tb0, ib0, semr0), (a1, tb1, ib1, semr1))
    xps = ((xpa, semoa), (xpb, semob))

    def idx_fetch(pair, iset):
        ia, it, ii, si = idxs[iset]
        off = base0 + pair * (2 * BLK)
        pltpu.make_async_copy(t0.at[pl.ds(off, 2 * BLK)], ia, si).start()
        pltpu.make_async_copy(t1.at[pl.ds(off, 2 * BLK)], it, si).start()
        pltpu.make_async_copy(t2.at[pl.ds(off, 2 * BLK)], ii, si).start()

    def idx_wait(iset):
        ia, it, ii, si = idxs[iset]
        pltpu.make_async_copy(t0.at[pl.ds(base0, 2 * BLK)], ia, si).wait()
        pltpu.make_async_copy(t0.at[pl.ds(base0, 2 * BLK)], it, si).wait()
        pltpu.make_async_copy(t0.at[pl.ds(base0, 2 * BLK)], ii, si).wait()

    H = BLK // 2

    def fire(rset, iset, half):
        ia, it, ii, _ = idxs[iset]
        ab, tb, ib, semr = rows[rset]
        sl0 = pl.ds(half * BLK, H)
        sl1 = pl.ds(half * BLK + H, H)
        pltpu.make_async_copy(emb.at[ia.at[sl0]], ab.at[pl.ds(0, H), :], semr).start()
        pltpu.make_async_copy(emb.at[it.at[sl0]], tb.at[pl.ds(0, H), :], semr).start()
        pltpu.make_async_copy(emb.at[ii.at[sl0]], ib.at[pl.ds(0, H), :], semr).start()
        pltpu.make_async_copy(emb.at[ia.at[sl1]], ab.at[pl.ds(H, H), :], semr).start()
        pltpu.make_async_copy(emb.at[it.at[sl1]], tb.at[pl.ds(H, H), :], semr).start()
        pltpu.make_async_copy(emb.at[ii.at[sl1]], ib.at[pl.ds(H, H), :], semr).start()

    def drain(rset):
        ia, _, _, _ = idxs[0]
        ab, tb, ib, semr = rows[rset]
        sl = pl.ds(0, H)
        for dst in (ab, tb, ib):
            pltpu.make_async_copy(
                emb.at[ia.at[sl]], dst.at[pl.ds(0, H), :], semr).wait()
            pltpu.make_async_copy(
                emb.at[ia.at[sl]], dst.at[pl.ds(H, H), :], semr).wait()

    def out_start(q, xset):
        xp, semo = xps[xset]
        off = (base0 + q * (GRP * BLK)) * LANES
        pltpu.make_async_copy(xp, out.at[pl.ds(off, GRP * BLK * LANES)], semo).start()

    def out_wait(xset):
        xp, semo = xps[xset]
        pltpu.make_async_copy(
            xp, out.at[pl.ds(base0 * LANES, GRP * BLK * LANES)], semo).wait()

    def compute(blk, rset, xset, xrow):
        ab, tb, ib, _ = rows[rset]
        xp, _ = xps[xset]

        def triplet(j, carry):
            acc1 = jnp.zeros((LANES,), jnp.float32)
            acc12 = jnp.zeros((LANES,), jnp.float32)
            acc2 = jnp.zeros((LANES,), jnp.float32)
            for c in range(CHUNKS):
                sl = pl.ds(c * LANES, LANES)
                av = ab[j, sl]
                tv = tb[j, sl]
                iv = ib[j, sl]
                d1 = tv - av
                d2 = iv - av
                acc1 = acc1 + d1 * d1
                acc12 = acc12 + d1 * d2
                acc2 = acc2 + d2 * d2
            xp[pl.ds((xrow + j) * LANES, LANES)] = 0.5 * acc1 + acc12 - acc2
            return carry

        lax.fori_loop(0, BLK, triplet, 0)

    def schedule(nb):
        npair = nb // 2
        idx_fetch(0, 0)
        idx_fetch(1, 1)
        idx_wait(0)
        fire(0, 0, 0)
        fire(1, 0, 1)
        drain(0)
        compute(0, 0, 0, 0 * BLK)
        idx_wait(1)
        fire(0, 1, 0)
        drain(1)
        idx_fetch(2, 0)
        compute(1, 1, 0, 1 * BLK)
        fire(1, 1, 1)
        drain(0)
        compute(2, 0, 0, 2 * BLK)
        idx_wait(0)
        fire(0, 0, 0)
        drain(1)
        idx_fetch(3, 1)
        compute(3, 1, 0, 3 * BLK)
        out_start(0, 0)

        def outer(t, carry):
            q1 = 2 * t + 1
            b0 = q1 * GRP

            @pl.when(t > 0)
            def _():
                out_wait(1)

            fire(1, 0, 1)
            drain(0)
            compute(b0 + 0, 0, 1, 0 * BLK)
            idx_wait(1)
            fire(0, 1, 0)
            drain(1)
            idx_fetch(4 * t + 4, 0)
            compute(b0 + 1, 1, 1, 1 * BLK)
            fire(1, 1, 1)
            drain(0)
            compute(b0 + 2, 0, 1, 2 * BLK)
            idx_wait(0)
            fire(0, 0, 0)
            drain(1)
            idx_fetch(4 * t + 5, 1)
            compute(b0 + 3, 1, 1, 3 * BLK)
            out_start(q1, 1)

            q2 = q1 + 1
            c0 = q2 * GRP
            out_wait(0)
            fire(1, 0, 1)
            drain(0)
            compute(c0 + 0, 0, 0, 0 * BLK)
            idx_wait(1)
            fire(0, 1, 0)
            drain(1)

            @pl.when(4 * t + 6 < npair)
            def _():
                idx_fetch(4 * t + 6, 0)

            compute(c0 + 1, 1, 0, 1 * BLK)
            fire(1, 1, 1)
            drain(0)
            compute(c0 + 2, 0, 0, 2 * BLK)
            @pl.when(c0 + 4 < nb)
            def _():
                idx_wait(0)
                fire(0, 0, 0)

            drain(1)

            @pl.when(4 * t + 7 < npair)
            def _():
                idx_fetch(4 * t + 7, 1)

            compute(c0 + 3, 1, 0, 3 * BLK)
            out_start(q2, 0)
            return carry

        lax.fori_loop(0, (nb // GRP - 1) // 2, outer, 0)
        out_wait(1)
        out_wait(0)

    @pl.when(cbit == 0)
    def _():
        schedule(NB_A)

    @pl.when(cbit == 1)
    def _():
        schedule(NB_B)


_TC_G = 4
_TC_R = N_PAD // 16 // _TC_G


def _tc_reduce_body(x_ref, out_ref):
    i = pl.program_id(0)
    x = x_ref[...].reshape(_TC_R, 256)
    g = lax.broadcasted_iota(jnp.int32, (256, 16), 0) // 16 == \
        lax.broadcasted_iota(jnp.int32, (256, 16), 1)
    s = jax.lax.dot_general(x, g.astype(jnp.float32),
                            (((1,), (0,)), ((), ())),
                            preferred_element_type=jnp.float32)
    sp = jnp.maximum(s, 0.0) + jnp.log1p(jnp.exp(-jnp.abs(s)))
    ids = (lax.broadcasted_iota(jnp.int32, (_TC_R, 16), 0) + i * _TC_R) * 16 + \
        lax.broadcasted_iota(jnp.int32, (_TC_R, 16), 1)
    part = jnp.sum(jnp.where(ids < N_TRIP, sp, 0.0),
                   keepdims=True).reshape(1, 1) / N_TRIP

    @pl.when(i == 0)
    def _():
        out_ref[...] = part

    @pl.when(i > 0)
    def _():
        out_ref[...] = out_ref[...] + part


_tc_reduce = pl.pallas_call(
    _tc_reduce_body,
    grid=(_TC_G,),
    in_specs=[pl.BlockSpec((N_PAD * 16 // _TC_G,), lambda i: (i,))],
    out_specs=pl.BlockSpec((1, 1), lambda i: (0, 0)),
    out_shape=jax.ShapeDtypeStruct((1, 1), jnp.float32),
)


def kernel(inputs, targets, T):
    del targets
    t_pad = jnp.pad(T, ((0, 0), (0, N_PAD - T.shape[1])))
    xpart = _sc_partials(inputs, t_pad[0], t_pad[1], t_pad[2])
    return _tc_reduce(xpart)[0, 0]
